# baseline (speedup 1.0000x reference)
import functools
import math

import jax
import jax.numpy as jnp
from jax import lax
from jax.experimental import pallas as pl
from jax.experimental.pallas import tpu as pltpu
from jax.experimental.pallas import tpu_sc as plsc

B = 4096
DIM = 128
CH = (12, 18, 20, 12)
CH0 = (0, 12, 30, 50)
NCLS = 62
NCH = 62

NC = 2
NS = 16
NW = NC * NS
ROWS_PER_W = B // NW
R = 4
NSLABS = ROWS_PER_W // R
NH = NSLABS // 2

CSC = 34
X2_SC = 12
X3_SC = 10

RUNS = (
    (0, 0, 1, 1),
    (0, 1, 0, 1),
    (0, 2, 2, 3),
    (1, 5, 0, 1),
    (0, 6, 5, 7),
    (1, 13, 1, 3),
    (2, 16, 0, 5),
    (1, 21, 4, 4),
    (2, 25, 5, 5),
    (1, 30, 8, 4),
    (2, 34, 10, 5),
    (1, 39, 12, 4),
    (2, 43, 15, 5),
    (1, 48, 16, 2),
    (3, 50, 0, 12),
)


_BT = 512


def _tc_body(l1, l2, l3, l4, x1, x2, x3, x4, wref, bref,
             o1, o2, o3, o4, ou):
    cols = []
    for ref in (l1, l2, l3, l4):
        l = ref[...]
        m = jnp.max(l, axis=1, keepdims=True)
        e = jnp.exp(l - m)
        z = jnp.sum(e, axis=1, keepdims=True)
        p = e / z
        maxp = jnp.max(p, axis=1, keepdims=True)
        ent = -jnp.sum(p * jnp.log(p + 1e-08), axis=1, keepdims=True)
        nent = ent * (1.0 / math.log(float(NCLS)))
        cols.append(maxp * (1.0 - nent))
    conf = jnp.concatenate(cols, axis=1)
    cm = jnp.max(conf, axis=1, keepdims=True)
    ce = jnp.exp(conf - cm)
    nc = ce / jnp.sum(ce, axis=1, keepdims=True)

    w = wref[...]
    b_all = bref[...]
    us = []
    for k, xref in enumerate((x1, x2, x3, x4)):
        xk = xref[...]
        wk = w[CH0[k]:CH0[k] + CH[k], :]
        d = jnp.sum(jnp.sum(xk * wk[:, None, :], axis=0), axis=1,
                    keepdims=True)
        ub = 1.0 / (1.0 + jnp.exp(-(d + b_all[:, k:k + 1])))
        us.append(ub * nc[:, k:k + 1])
    for o, k in ((o1, 0), (o2, 1), (o3, 2), (o4, 3)):
        o[...] = nc[:, k:k + 1]
    ou[...] = jnp.concatenate(us + [jnp.zeros((_BT, 12), jnp.float32)], axis=1)


def _tc_dense(l1, l2, l3, l4, xt1, xt2, xt3, xt4, w, bcat):
    lspec = pl.BlockSpec((_BT, NCLS), lambda i: (i, 0))
    xspecs = [pl.BlockSpec((CH[k], _BT, DIM), lambda i: (0, i, 0))
              for k in range(4)]
    colspec = pl.BlockSpec((_BT, 1), lambda i: (i, 0))
    return pl.pallas_call(
        _tc_body,
        grid=(B // _BT,),
        in_specs=[lspec] * 4 + xspecs + [
            pl.BlockSpec((NCH, DIM), lambda i: (0, 0)),
            pl.BlockSpec((1, 4), lambda i: (0, 0)),
        ],
        out_specs=[colspec] * 4 + [pl.BlockSpec((_BT, 16), lambda i: (i, 0))],
        out_shape=[jax.ShapeDtypeStruct((B, 1), jnp.float32)] * 4
        + [jax.ShapeDtypeStruct((B, 16), jnp.float32)],
    )(l1, l2, l3, l4, xt1, xt2, xt3, xt4, w, bcat)


_TC_RUNS = (
    (2, 0, 0, 5),
    (1, 5, 0, 4),
    (2, 9, 5, 5),
    (1, 14, 4, 2),
    (3, 16, 0, 12),
)

_BTE = 512


def _tce_body(x2, x3, x4, gref, uref, out):
    gv = gref[...]
    uv = uref[...]
    xs = (None, x2[...], x3[...], x4[...])
    for k, c0, p0, nrun in _TC_RUNS:
        uk = uv[:, k][None, :, None]
        out[c0:c0 + nrun] = (gv[c0:c0 + nrun] + xs[k][p0:p0 + nrun]) * uk


def _tc_elem(xt2, xt3, xt4, gtail, u16):
    return pl.pallas_call(
        _tce_body,
        grid=(B // _BTE,),
        in_specs=[
            pl.BlockSpec((6, _BTE, DIM), lambda i: (2, i, 0)),
            pl.BlockSpec((10, _BTE, DIM), lambda i: (1, i, 0)),
            pl.BlockSpec((CH[3], _BTE, DIM), lambda i: (0, i, 0)),
            pl.BlockSpec((NCH - CSC, _BTE, DIM), lambda i: (0, i, 0)),
            pl.BlockSpec((_BTE, 16), lambda i: (i, 0)),
        ],
        out_specs=pl.BlockSpec((NCH - CSC, _BTE, DIM), lambda i: (0, i, 0)),
        out_shape=jax.ShapeDtypeStruct((NCH - CSC, B, DIM), jnp.float32),
    )(xt2, xt3, xt4, gtail, u16)


def _lane_gather(v, idx):
    dn = lax.GatherDimensionNumbers(
        offset_dims=(), collapsed_slice_dims=(0,), start_index_map=(0,))
    return lax.gather(v, idx[:, None], dn, (1,),
                      mode=lax.GatherScatterMode.PROMISE_IN_BOUNDS)


def _splat(v, lane):
    return _lane_gather(v, jnp.full((16,), lane, jnp.int32))


_SC_RUNS = (
    (0, 0, 1, 1),
    (0, 1, 0, 1),
    (0, 2, 2, 3),
    (1, 5, 0, 1),
    (0, 6, 5, 7),
    (1, 13, 1, 3),
    (2, 16, 0, 5),
    (1, 21, 4, 4),
    (2, 25, 5, 5),
    (1, 30, 8, 4),
)


def _sc_body(x1, x2, x3, g, u16f, out, u_v,
             xa, ga, oa, sin_a, sout_a,
             xb, gb, ob, sin_b, sout_b):
    wid = lax.axis_index("s") * NC + lax.axis_index("c")
    row0 = wid * ROWS_PER_W

    pltpu.sync_copy(u16f.at[pl.ds(row0 * 16, ROWS_PER_W * 16)], u_v)

    xhbm = (x1, x2, x3)
    xch = (CH[0], X2_SC, X3_SC)

    def issue_in(s, xv, gv, sem):
        base = row0 + s * R
        for k in range(3):
            pltpu.async_copy(
                xhbm[k].at[pl.ds(0, xch[k]), pl.ds(base, R)], xv[k], sem)
        pltpu.async_copy(g.at[pl.ds(0, CSC), pl.ds(base, R)], gv, sem)

    def wait_in(xv, gv, sem):
        for k in range(3):
            pltpu.make_async_copy(
                xhbm[k].at[pl.ds(0, xch[k]), pl.ds(0, R)], xv[k], sem).wait()
        pltpu.make_async_copy(
            g.at[pl.ds(0, CSC), pl.ds(0, R)], gv, sem).wait()

    def issue_out(s, ov, sem):
        base = row0 + s * R
        pltpu.async_copy(ov, out.at[:, pl.ds(base, R)], sem)

    def wait_out(ov, sem):
        pltpu.make_async_copy(ov, out.at[:, pl.ds(0, R)], sem).wait()

    def compute(s, xv, gv, ov):
        us = []
        for r in range(R):
            u16 = u_v[pl.ds((s * R + r) * 16, 16)]
            us.append([_splat(u16, k) for k in range(3)])
        for k, c0, p0, nrun in _SC_RUNS:
            xref = xv[k]
            uks = [us[r][k] for r in range(R)]

            def run_body(ci, _, xref=xref, uks=uks, c0=c0, p0=p0,
                         gv=gv, ov=ov):
                for r in range(R):
                    for o in range(8):
                        sl = pl.ds(o * 16, 16)
                        ov[c0 + ci, r, sl] = (gv[c0 + ci, r, sl]
                                              + xref[p0 + ci, r, sl]) * uks[r]
                return 0

            lax.fori_loop(0, nrun, run_body, 0)

    issue_in(0, xa, ga, sin_a)
    issue_in(1, xb, gb, sin_b)

    def ring(sp, _):
        s0 = 2 * sp
        pl.when(sp > 0)(lambda: wait_out(oa, sout_a))
        wait_in(xa, ga, sin_a)
        compute(s0, xa, ga, oa)
        issue_out(s0, oa, sout_a)
        pl.when(sp < NH - 1)(lambda: issue_in(s0 + 2, xa, ga, sin_a))
        pl.when(sp > 0)(lambda: wait_out(ob, sout_b))
        wait_in(xb, gb, sin_b)
        compute(s0 + 1, xb, gb, ob)
        issue_out(s0 + 1, ob, sout_b)
        pl.when(sp < NH - 1)(lambda: issue_in(s0 + 3, xb, gb, sin_b))
        return 0

    lax.fori_loop(0, NH, ring, 0)
    wait_out(oa, sout_a)
    wait_out(ob, sout_b)


def _slab_scratch():
    return (
        [pltpu.VMEM((CH[0], R, DIM), jnp.float32),
         pltpu.VMEM((X2_SC, R, DIM), jnp.float32),
         pltpu.VMEM((X3_SC, R, DIM), jnp.float32),
         pltpu.VMEM((CSC, R, DIM), jnp.float32),
         pltpu.VMEM((CSC, R, DIM), jnp.float32),
         pltpu.SemaphoreType.DMA,
         pltpu.SemaphoreType.DMA]
    )


@functools.lru_cache(maxsize=1)
def _sc_trans():
    @functools.partial(
        pl.kernel,
        mesh=plsc.VectorSubcoreMesh(core_axis_name="c", subcore_axis_name="s"),
        out_type=jax.ShapeDtypeStruct((CSC, B, DIM), jnp.float32),
        scratch_types=[pltpu.VMEM((ROWS_PER_W * 16,), jnp.float32)]
        + _slab_scratch() + _slab_scratch(),
    )
    def sc(x1, x2, x3, g, u16f, out, u_v,
           xa1, xa2, xa3, ga, oa, sin_a, sout_a,
           xb1, xb2, xb3, gb, ob, sin_b, sout_b):
        _sc_body(x1, x2, x3, g, u16f, out, u_v,
                 (xa1, xa2, xa3), ga, oa, sin_a, sout_a,
                 (xb1, xb2, xb3), gb, ob, sin_b, sout_b)

    return sc


def kernel(x_1, x_2, x_3, x_4, g, logits_1, logits_2, logits_3, logits_4,
           W1, b1, W2, b2, W3, b3, W4, b4):
    xt = [jnp.transpose(x, (1, 0, 2)) for x in (x_1, x_2, x_3, x_4)]
    gt = jnp.transpose(g, (1, 0, 2))
    w = jnp.concatenate([
        W1.reshape(CH[0], DIM), W2.reshape(CH[1], DIM),
        W3.reshape(CH[2], DIM), W4.reshape(CH[3], DIM)], axis=0)
    bcat = jnp.concatenate([b1, b2, b3, b4]).astype(jnp.float32).reshape(1, 4)
    nc1, nc2, nc3, nc4, u16 = _tc_dense(
        logits_1, logits_2, logits_3, logits_4,
        xt[0], xt[1], xt[2], xt[3], w, bcat)
    ot_sc = _sc_trans()(xt[0], xt[1], xt[2], gt, u16.reshape(-1))
    ot_tc = _tc_elem(xt[1], xt[2], xt[3], lax.slice_in_dim(gt, CSC, NCH, axis=0),
                     u16)
    ot = jnp.concatenate([ot_sc, ot_tc], axis=0)
    return (nc1, nc2, nc3, nc4, jnp.transpose(ot, (1, 0, 2)))


# speedup vs baseline: 1.2528x; 1.2528x over previous
import functools
import math

import jax
import jax.numpy as jnp
from jax import lax
from jax.experimental import pallas as pl
from jax.experimental.pallas import tpu as pltpu
from jax.experimental.pallas import tpu_sc as plsc

B = 4096
DIM = 128
CH = (12, 18, 20, 12)
CH0 = (0, 12, 30, 50)
NCLS = 62
NCH = 62

NC = 2
NS = 16
NW = NC * NS
ROWS_PER_W = B // NW
R = 2
NSLABS = ROWS_PER_W // R
NH = NSLABS // 2

RUNS = (
    (0, 0, 1, 1),
    (0, 1, 0, 1),
    (0, 2, 2, 3),
    (1, 5, 0, 1),
    (0, 6, 5, 7),
    (1, 13, 1, 3),
    (2, 16, 0, 5),
    (1, 21, 4, 4),
    (2, 25, 5, 5),
    (1, 30, 8, 4),
    (2, 34, 10, 5),
    (1, 39, 12, 4),
    (2, 43, 15, 5),
    (1, 48, 16, 2),
    (3, 50, 0, 12),
)


_BT = 512


def _tc_body(l1, l2, l3, l4, x1, x2, x3, x4, wref, bref,
             o1, o2, o3, o4, ou):
    cols = []
    for ref in (l1, l2, l3, l4):
        l = ref[...]
        m = jnp.max(l, axis=1, keepdims=True)
        e = jnp.exp(l - m)
        z = jnp.sum(e, axis=1, keepdims=True)
        p = e / z
        maxp = jnp.max(p, axis=1, keepdims=True)
        ent = -jnp.sum(p * jnp.log(p + 1e-08), axis=1, keepdims=True)
        nent = ent * (1.0 / math.log(float(NCLS)))
        cols.append(maxp * (1.0 - nent))
    conf = jnp.concatenate(cols, axis=1)
    cm = jnp.max(conf, axis=1, keepdims=True)
    ce = jnp.exp(conf - cm)
    nc = ce / jnp.sum(ce, axis=1, keepdims=True)

    w = wref[...]
    b_all = bref[...]
    us = []
    for k, xref in enumerate((x1, x2, x3, x4)):
        xk = xref[...]
        wk = w[CH0[k]:CH0[k] + CH[k], :]
        d = jnp.sum(jnp.sum(xk * wk[:, None, :], axis=0), axis=1,
                    keepdims=True)
        ub = 1.0 / (1.0 + jnp.exp(-(d + b_all[:, k:k + 1])))
        us.append(ub * nc[:, k:k + 1])
    for o, k in ((o1, 0), (o2, 1), (o3, 2), (o4, 3)):
        o[...] = nc[:, k:k + 1]
    ou[...] = jnp.concatenate(us + [jnp.zeros((_BT, 12), jnp.float32)], axis=1)


def _tc_dense(l1, l2, l3, l4, xt1, xt2, xt3, xt4, w, bcat):
    lspec = pl.BlockSpec((_BT, NCLS), lambda i: (i, 0))
    xspecs = [pl.BlockSpec((CH[k], _BT, DIM), lambda i: (0, i, 0))
              for k in range(4)]
    colspec = pl.BlockSpec((_BT, 1), lambda i: (i, 0))
    return pl.pallas_call(
        _tc_body,
        grid=(B // _BT,),
        in_specs=[lspec] * 4 + xspecs + [
            pl.BlockSpec((NCH, DIM), lambda i: (0, 0)),
            pl.BlockSpec((1, 4), lambda i: (0, 0)),
        ],
        out_specs=[colspec] * 4 + [pl.BlockSpec((_BT, 16), lambda i: (i, 0))],
        out_shape=[jax.ShapeDtypeStruct((B, 1), jnp.float32)] * 4
        + [jax.ShapeDtypeStruct((B, 16), jnp.float32)],
    )(l1, l2, l3, l4, xt1, xt2, xt3, xt4, w, bcat)


def _lane_gather(v, idx):
    dn = lax.GatherDimensionNumbers(
        offset_dims=(), collapsed_slice_dims=(0,), start_index_map=(0,))
    return lax.gather(v, idx[:, None], dn, (1,),
                      mode=lax.GatherScatterMode.PROMISE_IN_BOUNDS)


def _splat(v, lane):
    return _lane_gather(v, jnp.full((16,), lane, jnp.int32))


def _sc_body(x1, x2, x3, x4, g, u16f, out, u_v,
             xa, ga, oa, sin_a, sout_a,
             xb, gb, ob, sin_b, sout_b):
    wid = lax.axis_index("s") * NC + lax.axis_index("c")
    row0 = wid * ROWS_PER_W

    pltpu.sync_copy(u16f.at[pl.ds(row0 * 16, ROWS_PER_W * 16)], u_v)

    xhbm = (x1, x2, x3, x4)

    def issue_in(s, xv, gv, sem):
        base = row0 + s * R
        for k in range(4):
            pltpu.async_copy(xhbm[k].at[:, pl.ds(base, R)], xv[k], sem)
        pltpu.async_copy(g.at[:, pl.ds(base, R)], gv, sem)

    def wait_in(xv, gv, sem):
        for k in range(4):
            pltpu.make_async_copy(
                xhbm[k].at[:, pl.ds(0, R)], xv[k], sem).wait()
        pltpu.make_async_copy(g.at[:, pl.ds(0, R)], gv, sem).wait()

    def issue_out(s, ov, sem):
        base = row0 + s * R
        pltpu.async_copy(ov, out.at[:, pl.ds(base, R)], sem)

    def wait_out(ov, sem):
        pltpu.make_async_copy(ov, out.at[:, pl.ds(0, R)], sem).wait()

    def compute(s, xv, gv, ov):
        us = []
        for r in range(R):
            u16 = u_v[pl.ds((s * R + r) * 16, 16)]
            us.append([_splat(u16, k) for k in range(4)])
        for k, c0, p0, nrun in RUNS:
            xref = xv[k]
            uks = [us[r][k] for r in range(R)]

            def run_body(ci, _, xref=xref, uks=uks, c0=c0, p0=p0,
                         gv=gv, ov=ov):
                for r in range(R):
                    for o in range(8):
                        sl = pl.ds(o * 16, 16)
                        ov[c0 + ci, r, sl] = (gv[c0 + ci, r, sl]
                                              + xref[p0 + ci, r, sl]) * uks[r]
                return 0

            lax.fori_loop(0, nrun, run_body, 0)

    issue_in(0, xa, ga, sin_a)
    issue_in(1, xb, gb, sin_b)

    def ring(sp, _):
        s0 = 2 * sp
        pl.when(sp > 0)(lambda: wait_out(oa, sout_a))
        wait_in(xa, ga, sin_a)
        compute(s0, xa, ga, oa)
        issue_out(s0, oa, sout_a)
        pl.when(sp < NH - 1)(lambda: issue_in(s0 + 2, xa, ga, sin_a))
        pl.when(sp > 0)(lambda: wait_out(ob, sout_b))
        wait_in(xb, gb, sin_b)
        compute(s0 + 1, xb, gb, ob)
        issue_out(s0 + 1, ob, sout_b)
        pl.when(sp < NH - 1)(lambda: issue_in(s0 + 3, xb, gb, sin_b))
        return 0

    lax.fori_loop(0, NH, ring, 0)
    wait_out(oa, sout_a)
    wait_out(ob, sout_b)


def _slab_scratch():
    return (
        [pltpu.VMEM((CH[k], R, DIM), jnp.float32) for k in range(4)]
        + [pltpu.VMEM((NCH, R, DIM), jnp.float32),
           pltpu.VMEM((NCH, R, DIM), jnp.float32),
           pltpu.SemaphoreType.DMA,
           pltpu.SemaphoreType.DMA]
    )


@functools.lru_cache(maxsize=1)
def _sc_trans():
    @functools.partial(
        pl.kernel,
        mesh=plsc.VectorSubcoreMesh(core_axis_name="c", subcore_axis_name="s"),
        out_type=jax.ShapeDtypeStruct((NCH, B, DIM), jnp.float32),
        scratch_types=[pltpu.VMEM((ROWS_PER_W * 16,), jnp.float32)]
        + _slab_scratch() + _slab_scratch(),
    )
    def sc(x1, x2, x3, x4, g, u16f, out, u_v,
           xa1, xa2, xa3, xa4, ga, oa, sin_a, sout_a,
           xb1, xb2, xb3, xb4, gb, ob, sin_b, sout_b):
        _sc_body(x1, x2, x3, x4, g, u16f, out, u_v,
                 (xa1, xa2, xa3, xa4), ga, oa, sin_a, sout_a,
                 (xb1, xb2, xb3, xb4), gb, ob, sin_b, sout_b)

    return sc


def kernel(x_1, x_2, x_3, x_4, g, logits_1, logits_2, logits_3, logits_4,
           W1, b1, W2, b2, W3, b3, W4, b4):
    xt = [jnp.transpose(x, (1, 0, 2)) for x in (x_1, x_2, x_3, x_4)]
    gt = jnp.transpose(g, (1, 0, 2))
    w = jnp.concatenate([
        W1.reshape(CH[0], DIM), W2.reshape(CH[1], DIM),
        W3.reshape(CH[2], DIM), W4.reshape(CH[3], DIM)], axis=0)
    bcat = jnp.concatenate([b1, b2, b3, b4]).astype(jnp.float32).reshape(1, 4)
    nc1, nc2, nc3, nc4, u16 = _tc_dense(
        logits_1, logits_2, logits_3, logits_4,
        xt[0], xt[1], xt[2], xt[3], w, bcat)
    ot = _sc_trans()(xt[0], xt[1], xt[2], xt[3], gt, u16.reshape(-1))
    return (nc1, nc2, nc3, nc4, jnp.transpose(ot, (1, 0, 2)))
